# baseline (speedup 1.0000x reference)
import jax
import jax.numpy as jnp
from jax.experimental import pallas as pl
from jax.experimental.pallas import tpu as pltpu


def _vq_kernel(z_ref, dic_ref, out_ref, d2_ref):
    z = z_ref[...]
    dic = dic_ref[...]
    n = z.shape[0]
    k = dic.shape[0]

    @pl.when(pl.program_id(0) == 0)
    def _():
        ones = jnp.ones((1, z.shape[1]), jnp.float32)
        d2_ref[...] = jax.lax.dot_general(
            ones, dic * dic, (((1,), (1,)), ((), ())),
            precision=jax.lax.Precision.HIGHEST,
            preferred_element_type=jnp.float32)

    cross = jax.lax.dot_general(
        z, dic, (((1,), (1,)), ((), ())),
        precision=jax.lax.Precision.DEFAULT,
        preferred_element_type=jnp.float32)
    z2 = jnp.sum(z * z, axis=1, keepdims=True)
    dist = (d2_ref[...] + z2) - 2.0 * cross
    minval = jnp.min(dist, axis=1, keepdims=True)
    iota = jax.lax.broadcasted_iota(jnp.int32, (n, k), 1)
    idx = jnp.min(jnp.where(dist == minval, iota, k), axis=1, keepdims=True)
    onehot = (iota == idx).astype(jnp.float32)
    out_ref[...] = jax.lax.dot_general(
        onehot, dic, (((1,), (0,)), ((), ())),
        precision=jax.lax.Precision.DEFAULT,
        preferred_element_type=jnp.float32)


def kernel(ze, dictionary):
    b, t, d = ze.shape
    n = b * t
    k = dictionary.shape[0]
    z = ze.reshape(n, d)
    blk = 1024
    out = pl.pallas_call(
        _vq_kernel,
        grid=(n // blk,),
        in_specs=[
            pl.BlockSpec((blk, d), lambda i: (i, 0)),
            pl.BlockSpec((k, d), lambda i: (0, 0)),
        ],
        out_specs=pl.BlockSpec((blk, d), lambda i: (i, 0)),
        out_shape=jax.ShapeDtypeStruct((n, d), jnp.float32),
        scratch_shapes=[pltpu.VMEM((1, k), jnp.float32)],
    )(z, dictionary)
    return out.reshape(b, t, d)


# speedup vs baseline: 1.0146x; 1.0146x over previous
import jax
import jax.numpy as jnp
from jax.experimental import pallas as pl
from jax.experimental.pallas import tpu as pltpu


def _vq_kernel(z_ref, dic_ref, out_ref, d2_ref):
    z = z_ref[...]
    dic = dic_ref[...]
    n = z.shape[0]
    k = dic.shape[0]

    @pl.when(pl.program_id(0) == 0)
    def _():
        ones = jnp.ones((1, z.shape[1]), jnp.float32)
        d2_ref[...] = jax.lax.dot_general(
            ones, dic * dic, (((1,), (1,)), ((), ())),
            precision=jax.lax.Precision.HIGHEST,
            preferred_element_type=jnp.float32)

    cross = jax.lax.dot_general(
        z, dic, (((1,), (1,)), ((), ())),
        precision=jax.lax.Precision.DEFAULT,
        preferred_element_type=jnp.float32)
    z2 = jnp.sum(z * z, axis=1, keepdims=True)
    dist = (d2_ref[...] + z2) - 2.0 * cross
    minval = jnp.min(dist, axis=1, keepdims=True)
    iota = jax.lax.broadcasted_iota(jnp.int32, (n, k), 1)
    idx = jnp.min(jnp.where(dist == minval, iota, k), axis=1, keepdims=True)
    onehot = (iota == idx).astype(jnp.float32)
    out_ref[...] = jax.lax.dot_general(
        onehot, dic, (((1,), (0,)), ((), ())),
        precision=jax.lax.Precision.DEFAULT,
        preferred_element_type=jnp.float32)


def kernel(ze, dictionary):
    b, t, d = ze.shape
    n = b * t
    k = dictionary.shape[0]
    z = ze.reshape(n, d)
    blk = 2048
    out = pl.pallas_call(
        _vq_kernel,
        grid=(n // blk,),
        in_specs=[
            pl.BlockSpec((blk, d), lambda i: (i, 0)),
            pl.BlockSpec((k, d), lambda i: (0, 0)),
        ],
        out_specs=pl.BlockSpec((blk, d), lambda i: (i, 0)),
        out_shape=jax.ShapeDtypeStruct((n, d), jnp.float32),
        scratch_shapes=[pltpu.VMEM((1, k), jnp.float32)],
    )(z, dictionary)
    return out.reshape(b, t, d)
